# baseline (speedup 1.0000x reference)
import functools

import jax
import jax.numpy as jnp
from jax import lax
from jax.experimental import pallas as pl
from jax.experimental.pallas import tpu as pltpu
from jax.experimental.pallas import tpu_sc as plsc

DIM = 32
BATCH = 16384
FIELDS = 26

NC = 2
NS = 16
NW = NC * NS
B = BATCH * FIELDS
BPW = B // NW
G = 128
CHUNK = 1664
KG = CHUNK // G
NCHUNK = BPW // CHUNK


def _sc_gather(idx, table):
    mesh = plsc.VectorSubcoreMesh(core_axis_name="c", subcore_axis_name="s")

    scratch = [
        pltpu.VMEM((CHUNK,), jnp.int32),
        pltpu.VMEM((CHUNK,), jnp.int32),
        pltpu.VMEM((CHUNK, DIM), jnp.float32),
        pltpu.VMEM((CHUNK, DIM), jnp.float32),
        pltpu.SemaphoreType.DMA,
        pltpu.SemaphoreType.DMA,
        pltpu.SemaphoreType.DMA,
        pltpu.SemaphoreType.DMA,
    ]

    @functools.partial(
        pl.kernel,
        mesh=mesh,
        out_type=jax.ShapeDtypeStruct((B, DIM), jnp.float32),
        scratch_types=scratch,
        compiler_params=pltpu.CompilerParams(use_tc_tiling_on_sc=False),
    )
    def k(idx_hbm, table_hbm, out_hbm, idx_v0, idx_v1, rows_v0, rows_v1,
          sem_g0, sem_g1, sem_w0, sem_w1):
        wid = lax.axis_index("s") * NC + lax.axis_index("c")
        base = wid * BPW
        idx_v = (idx_v0, idx_v1)
        rows_v = (rows_v0, rows_v1)
        sem_g = (sem_g0, sem_g1)
        sem_w = (sem_w0, sem_w1)
        wb = {}
        for c in range(NCHUNK):
            p = c % 2
            off = base + c * CHUNK
            if c >= 2:
                wb[c - 2].wait()
            pltpu.sync_copy(idx_hbm.at[pl.ds(off, CHUNK)], idx_v[p])
            copies = []
            for j in range(KG):
                cp = pltpu.make_async_copy(
                    table_hbm.at[idx_v[p].at[pl.ds(j * G, G)]],
                    rows_v[p].at[pl.ds(j * G, G)],
                    sem_g[p],
                )
                cp.start()
                copies.append(cp)
            for cp in copies:
                cp.wait()
            w = pltpu.make_async_copy(
                rows_v[p], out_hbm.at[pl.ds(off, CHUNK)], sem_w[p])
            w.start()
            wb[c] = w
        wb[NCHUNK - 2].wait()
        wb[NCHUNK - 1].wait()

    return k(idx, table)


def kernel(x, embedding):
    idx = x.reshape(-1).astype(jnp.int32)
    out = _sc_gather(idx, embedding)
    return out.reshape(BATCH, FIELDS, DIM)


# speedup vs baseline: 1.0051x; 1.0051x over previous
import functools

import jax
import jax.numpy as jnp
from jax import lax
from jax.experimental import pallas as pl
from jax.experimental.pallas import tpu as pltpu
from jax.experimental.pallas import tpu_sc as plsc

DIM = 32
BATCH = 16384
FIELDS = 26

NC = 2
NS = 16
NW = NC * NS
B = BATCH * FIELDS
BPW = B // NW
G = 128
CHUNK = 1664
KG = CHUNK // G
NCHUNK = BPW // CHUNK


def _sc_gather(idx, table):
    mesh = plsc.VectorSubcoreMesh(core_axis_name="c", subcore_axis_name="s")

    scratch = [
        pltpu.VMEM((BPW,), jnp.int32),
        pltpu.VMEM((CHUNK, DIM), jnp.float32),
        pltpu.VMEM((CHUNK, DIM), jnp.float32),
        pltpu.SemaphoreType.DMA,
        pltpu.SemaphoreType.DMA,
        pltpu.SemaphoreType.DMA,
        pltpu.SemaphoreType.DMA,
    ]

    @functools.partial(
        pl.kernel,
        mesh=mesh,
        out_type=jax.ShapeDtypeStruct((B, DIM), jnp.float32),
        scratch_types=scratch,
        compiler_params=pltpu.CompilerParams(use_tc_tiling_on_sc=False),
    )
    def k(idx_hbm, table_hbm, out_hbm, idx_v, rows_v0, rows_v1,
          sem_g0, sem_g1, sem_w0, sem_w1):
        wid = lax.axis_index("s") * NC + lax.axis_index("c")
        base = wid * BPW
        rows_v = (rows_v0, rows_v1)
        sem_g = (sem_g0, sem_g1)
        sem_w = (sem_w0, sem_w1)

        pltpu.sync_copy(idx_hbm.at[pl.ds(base, BPW)], idx_v)

        gathers = {}
        wb = {}
        for c in range(NCHUNK):
            p = c % 2
            if c >= 2:
                wb[c - 2].wait()
            copies = []
            for j in range(KG):
                cp = pltpu.make_async_copy(
                    table_hbm.at[idx_v.at[pl.ds(c * CHUNK + j * G, G)]],
                    rows_v[p].at[pl.ds(j * G, G)],
                    sem_g[p],
                )
                cp.start()
                copies.append(cp)
            gathers[c] = copies
            if c >= 1:
                for cp in gathers[c - 1]:
                    cp.wait()
                w = pltpu.make_async_copy(
                    rows_v[1 - p],
                    out_hbm.at[pl.ds(base + (c - 1) * CHUNK, CHUNK)],
                    sem_w[1 - p],
                )
                w.start()
                wb[c - 1] = w
        last = NCHUNK - 1
        for cp in gathers[last]:
            cp.wait()
        w = pltpu.make_async_copy(
            rows_v[last % 2],
            out_hbm.at[pl.ds(base + last * CHUNK, CHUNK)],
            sem_w[last % 2],
        )
        w.start()
        wb[last] = w
        wb[last - 1].wait()
        wb[last].wait()

    return k(idx, table)


def kernel(x, embedding):
    idx = x.reshape(-1).astype(jnp.int32)
    out = _sc_gather(idx, embedding)
    return out.reshape(BATCH, FIELDS, DIM)


# speedup vs baseline: 1.0325x; 1.0273x over previous
import functools

import jax
import jax.numpy as jnp
from jax import lax
from jax.experimental import pallas as pl
from jax.experimental.pallas import tpu as pltpu
from jax.experimental.pallas import tpu_sc as plsc

DIM = 32
BATCH = 16384
FIELDS = 26

NC = 2
NS = 16
NW = NC * NS
B = BATCH * FIELDS
G = 128
NBLK = B // G
BPW = NBLK // NW
NBUF = 4
TCD = BATCH // G


def _sc_gather(idx, table):
    mesh = plsc.VectorSubcoreMesh(core_axis_name="c", subcore_axis_name="s")

    scratch = (
        [pltpu.VMEM((BPW * G,), jnp.int32)]
        + [pltpu.VMEM((G, DIM), jnp.float32) for _ in range(NBUF)]
        + [pltpu.VMEM((G * DIM,), jnp.float32) for _ in range(NBUF)]
        + [pltpu.SemaphoreType.DMA] * (2 * NBUF)
    )

    @functools.partial(
        pl.kernel,
        mesh=mesh,
        out_type=jax.ShapeDtypeStruct((FIELDS * DIM * BATCH,), jnp.float32),
        scratch_types=scratch,
        compiler_params=pltpu.CompilerParams(
            use_tc_tiling_on_sc=False, needs_layout_passes=False),
    )
    def k(idx_hbm, table_hbm, out_hbm, idx_v, *bufs):
        rows = bufs[:NBUF]
        rowsT = bufs[NBUF:2 * NBUF]
        sem_g = bufs[2 * NBUF:3 * NBUF]
        sem_w = bufs[3 * NBUF:4 * NBUF]

        wid = lax.axis_index("s") * NC + lax.axis_index("c")
        blk0 = wid * BPW

        pltpu.sync_copy(idx_hbm.at[pl.ds(wid * BPW * G, BPW * G)], idx_v)

        iv128 = [(lax.iota(jnp.int32, 16) + h * 16) * G for h in range(2)]

        def gather(k_, u):
            return pltpu.make_async_copy(
                table_hbm.at[idx_v.at[pl.ds(k_ * G, G)]], rows[u], sem_g[u])

        def write(k_, u):
            blk = blk0 + k_
            f = blk // TCD
            tc = blk % TCD
            base = f * (DIM * BATCH) + tc * (8 * G)
            cps = []
            for tr in range(DIM // 8):
                cps.append(pltpu.make_async_copy(
                    rowsT[u].at[pl.ds(tr * (8 * G), 8 * G)],
                    out_hbm.at[pl.ds(base + tr * (8 * G * TCD), 8 * G)],
                    sem_w[u]))
            return cps

        def transpose(u):
            rows_u = rows[u]
            rowsT_u = rowsT[u]

            @pl.loop(0, G)
            def _(c):
                csplat = jnp.full((16,), c, jnp.int32)
                for h in range(2):
                    v = rows_u[c, pl.ds(h * 16, 16)]
                    plsc.store_scatter(rowsT_u, [iv128[h] + csplat], v)

        def start_writes(k_, u):
            for cp in write(k_, u):
                cp.start()

        def wait_writes(k_, u):
            for cp in write(k_, u):
                cp.wait()

        for u in range(NBUF):
            gather(u, u).start()

        for u in range(NBUF):
            gather(u, u).wait()
            transpose(u)
            start_writes(u, u)
            gather(u + NBUF, u).start()

        @pl.loop(NBUF, BPW - NBUF, step=NBUF)
        def _(i):
            for u in range(NBUF):
                k_ = i + u
                gather(k_, u).wait()
                wait_writes(k_ - NBUF, u)
                transpose(u)
                start_writes(k_, u)
                gather(k_ + NBUF, u).start()

        for u in range(NBUF):
            k_ = BPW - NBUF + u
            gather(k_, u).wait()
            wait_writes(k_ - NBUF, u)
            transpose(u)
            start_writes(k_, u)
        for u in range(NBUF):
            wait_writes(BPW - NBUF + u, u)

    return k(idx, table)


def kernel(x, embedding):
    idx = x.T.astype(jnp.int32).reshape(-1)
    flat = _sc_gather(idx, embedding)
    out5d = flat.reshape(FIELDS, DIM // 8, TCD, 8, G)
    return out5d.transpose(2, 4, 0, 1, 3).reshape(BATCH, FIELDS, DIM)
